# Initial kernel scaffold; baseline (speedup 1.0000x reference)
import jax
import jax.numpy as jnp
from jax.experimental import pallas as pl


def kernel(x, edge_index, batch, W1, b1, fc1_w, fc1_b, fc2_w, fc2_b, norm1_w, norm1_b, W2, b2, fc3_w, fc3_b, fc4_w, fc4_b, norm2_w, norm2_b, fcf_w, fcf_b):
    raise NotImplementedError("write your pallas kernel here")


# speedup vs baseline: 14.1327x; 14.1327x over previous
import functools
import jax
import jax.numpy as jnp
from jax import lax
from jax.experimental import pallas as pl
from jax.experimental.pallas import tpu as pltpu
from jax.experimental.pallas import tpu_sc as plsc

N = 50000
E = 800000
H = 64
G = 128
EPS = 1e-5

NPAD = 51200
EP = 819200
EROWS = EP // 128
DUMMY = NPAD - 1
BN = 2048
GRID = NPAD // BN
STRIPE = NPAD // 16
TROWS = EROWS // 32
CNT_EL = float(N * H)

_mesh = plsc.VectorSubcoreMesh(core_axis_name="c", subcore_axis_name="s")
_sc_params = pltpu.CompilerParams(use_tc_tiling_on_sc=False)


@functools.partial(
    pl.kernel,
    out_type=jax.ShapeDtypeStruct((2, NPAD), jnp.float32),
    mesh=_mesh,
    compiler_params=_sc_params,
    scratch_types=[
        pltpu.VMEM((TROWS, 128), jnp.int32),
        pltpu.VMEM((128,), jnp.float32),
        pltpu.VMEM_SHARED((NPAD,), jnp.float32),
    ],
)
def _deg_kernel(dst_hbm, zeros_hbm, out_hbm, dst_v, ones_v, acc):
    c = lax.axis_index("c")
    s = lax.axis_index("s")
    w = c * 16 + s
    pltpu.sync_copy(zeros_hbm, acc.at[pl.ds(s * STRIPE, STRIPE)])
    for i in range(8):
        ones_v[pl.ds(i * 16, 16)] = jnp.ones((16,), jnp.float32)
    pltpu.sync_copy(dst_hbm.at[pl.ds(w * TROWS, TROWS)], dst_v)
    plsc.subcore_barrier()

    def body(j, carry):
        pltpu.sync_copy(ones_v, acc.at[dst_v.at[j]], add=True)
        return carry

    lax.fori_loop(0, TROWS, body, 0)
    plsc.subcore_barrier()
    pltpu.sync_copy(acc.at[pl.ds(s * STRIPE, STRIPE)],
                    out_hbm.at[c, pl.ds(s * STRIPE, STRIPE)])


@functools.partial(
    pl.kernel,
    out_type=jax.ShapeDtypeStruct((2, NPAD, 16), jnp.float32),
    mesh=_mesh,
    compiler_params=_sc_params,
    scratch_types=[
        pltpu.VMEM((TROWS, 128), jnp.int32),
        pltpu.VMEM((TROWS, 128), jnp.int32),
        pltpu.VMEM((128, 16), jnp.float32),
        pltpu.VMEM((128, 16), jnp.float32),
        pltpu.VMEM_SHARED((NPAD, 16), jnp.float32),
        pltpu.SemaphoreType.DMA,
    ],
)
def _agg16_kernel(src_hbm, dst_hbm, tab_hbm, zrow_hbm, out_hbm,
                  src_v, dst_v, rows_v, zbuf, acc, sem):
    c = lax.axis_index("c")
    s = lax.axis_index("s")
    w = c * 16 + s
    pltpu.sync_copy(zrow_hbm, zbuf)

    def zb(k, carry):
        pltpu.sync_copy(zbuf, acc.at[pl.ds(s * STRIPE + k * 128, 128)])
        return carry

    lax.fori_loop(0, STRIPE // 128, zb, 0)
    pltpu.sync_copy(src_hbm.at[pl.ds(w * TROWS, TROWS)], src_v)
    pltpu.sync_copy(dst_hbm.at[pl.ds(w * TROWS, TROWS)], dst_v)
    plsc.subcore_barrier()

    def body(j, carry):
        pltpu.async_copy(tab_hbm.at[src_v.at[j]], rows_v, sem).wait()
        pltpu.sync_copy(rows_v, acc.at[dst_v.at[j]], add=True)
        return carry

    lax.fori_loop(0, TROWS, body, 0)
    plsc.subcore_barrier()
    pltpu.sync_copy(acc.at[pl.ds(s * STRIPE, STRIPE)],
                    out_hbm.at[c, pl.ds(s * STRIPE, STRIPE)])


def _k2_body(x_ref, d0_ref, d1_ref, dinv_ref, xs_ref):
    deg = d0_ref[0, 0, :] + d1_ref[0, 0, :] + 1.0
    dv = lax.rsqrt(deg).reshape(BN, 1)
    dinv_ref[...] = dv
    xs_ref[...] = x_ref[...] * dv


def _stats_vec(i, blk):
    rows = i * BN + lax.broadcasted_iota(jnp.int32, (BN, 1), 0)
    m = rows < N
    bm = jnp.where(m, blk, 0.0)
    s1 = jnp.sum(bm)
    s2 = jnp.sum(bm * bm)
    col = lax.broadcasted_iota(jnp.int32, (1, 128), 1)
    return jnp.where(col == 0, s1, 0.0) + jnp.where(col == 1, s2, 0.0)


def _k4_body(a0_ref, a1_ref, xs_ref, dinv_ref, w1_ref, b1_ref,
             f1w_ref, f1b_ref, f2w_ref, f2b_ref, h2_ref, st_ref):
    i = pl.program_id(0)
    pre = (a0_ref[0] + a1_ref[0] + xs_ref[...]) * dinv_ref[...]
    h1 = jax.nn.relu(jnp.dot(pre, w1_ref[...], preferred_element_type=jnp.float32)
                     + b1_ref[...])
    h = jax.nn.relu(jnp.dot(h1, f1w_ref[...], preferred_element_type=jnp.float32)
                    + f1b_ref[...])
    h2 = jax.nn.relu(jnp.dot(h, f2w_ref[...], preferred_element_type=jnp.float32)
                     + f2b_ref[...] + h1)
    h2_ref[...] = h2
    vec = _stats_vec(i, h2)

    @pl.when(i == 0)
    def _():
        st_ref[...] = vec

    @pl.when(i > 0)
    def _():
        st_ref[...] = st_ref[...] + vec


def _norm_consts(st_ref):
    st = st_ref[...]
    col = lax.broadcasted_iota(jnp.int32, (1, 128), 1)
    s1 = jnp.sum(jnp.where(col == 0, st, 0.0))
    s2 = jnp.sum(jnp.where(col == 1, st, 0.0))
    mean = s1 / CNT_EL
    var = s2 / CNT_EL - mean * mean
    scale = 1.0 / (jnp.sqrt(jnp.maximum(var, 0.0)) + EPS)
    return mean, scale


def _k5_body(h2_ref, st_ref, nw_ref, nb_ref, dinv_ref,
             hn_ref, g0_ref, g1_ref, g2_ref, g3_ref):
    mean, scale = _norm_consts(st_ref)
    hn = (h2_ref[...] - mean) * scale * nw_ref[...] + nb_ref[...]
    hn_ref[...] = hn
    g = hn * dinv_ref[...]
    g0_ref[...] = g[:, 0:16]
    g1_ref[...] = g[:, 16:32]
    g2_ref[...] = g[:, 32:48]
    g3_ref[...] = g[:, 48:64]


def _k7_body(a00_ref, a01_ref, a10_ref, a11_ref, a20_ref, a21_ref,
             a30_ref, a31_ref, hn_ref, dinv_ref, w2_ref, b2_ref,
             f3w_ref, f3b_ref, f4w_ref, f4b_ref, v_ref, st_ref):
    i = pl.program_id(0)
    aggfull = jnp.concatenate(
        [a00_ref[0] + a01_ref[0], a10_ref[0] + a11_ref[0],
         a20_ref[0] + a21_ref[0], a30_ref[0] + a31_ref[0]], axis=1)
    dv = dinv_ref[...]
    pre = (aggfull + hn_ref[...] * dv) * dv
    t = jnp.dot(pre, w2_ref[...], preferred_element_type=jnp.float32) + b2_ref[...]
    u = jax.nn.relu(jnp.dot(t, f3w_ref[...], preferred_element_type=jnp.float32)
                    + f3b_ref[...])
    v = (jnp.dot(u, f4w_ref[...], preferred_element_type=jnp.float32)
         + f4b_ref[...] + hn_ref[...])
    v_ref[...] = v
    vec = _stats_vec(i, v)

    @pl.when(i == 0)
    def _():
        st_ref[...] = vec

    @pl.when(i > 0)
    def _():
        st_ref[...] = st_ref[...] + vec


def _k8_body(v_ref, st_ref, nw_ref, nb_ref, fw_ref, fb_ref, b_ref,
             out_ref, acc_ref):
    i = pl.program_id(0)
    mean, scale = _norm_consts(st_ref)
    vn = (v_ref[...] - mean) * scale * nw_ref[...] + nb_ref[...]
    y = jnp.dot(vn, fw_ref[...], preferred_element_type=jnp.float32) + fb_ref[...]
    cols = lax.broadcasted_iota(jnp.int32, (BN, G), 1)
    oh = (b_ref[...] == cols).astype(jnp.float32)
    dn = (((0,), (0,)), ((), ()))
    sums_p = lax.dot_general(oh, y, dn, preferred_element_type=jnp.float32)
    cnt_p = lax.dot_general(oh, jnp.ones((BN, 1), jnp.float32), dn,
                            preferred_element_type=jnp.float32)
    part = jnp.concatenate([sums_p, cnt_p], axis=1)

    @pl.when(i == 0)
    def _():
        acc_ref[...] = part

    @pl.when(i > 0)
    def _():
        acc_ref[...] = acc_ref[...] + part

    @pl.when(i == GRID - 1)
    def _():
        out_ref[...] = acc_ref[:, :1] / jnp.maximum(acc_ref[:, 1:2], 1.0)


def _row_spec(cols):
    return pl.BlockSpec((BN, cols), lambda i: (i, 0))


def _const_spec(shape):
    nd = len(shape)
    return pl.BlockSpec(shape, lambda i: (0,) * nd)


def _part_spec(cols, core):
    return pl.BlockSpec((1, BN, cols), lambda i, c=core: (c, i, 0))


def kernel(x, edge_index, batch, W1, b1, fc1_w, fc1_b, fc2_w, fc2_b,
           norm1_w, norm1_b, W2, b2, fc3_w, fc3_b, fc4_w, fc4_b,
           norm2_w, norm2_b, fcf_w, fcf_b):
    f32 = jnp.float32
    src = edge_index[0]
    dst = edge_index[1]
    srcp = jnp.concatenate([src, jnp.zeros((EP - E,), jnp.int32)]).reshape(EROWS, 128)
    dstp = jnp.concatenate([dst, jnp.full((EP - E,), DUMMY, jnp.int32)]).reshape(EROWS, 128)
    x16 = jnp.pad(x, ((0, NPAD - N), (0, 16 - 9)))
    batchp = jnp.concatenate([batch, jnp.full((NPAD - N,), G, jnp.int32)]).reshape(NPAD, 1)

    z_stripe = jnp.zeros((STRIPE,), f32)
    z16 = jnp.zeros((128, 16), f32)

    w1p = jnp.pad(W1, ((0, 7), (0, 0)))
    b1r = b1.reshape(1, H)
    f1br = fc1_b.reshape(1, H)
    f2br = fc2_b.reshape(1, H)
    n1wr = norm1_w.reshape(1, H)
    n1br = norm1_b.reshape(1, H)
    b2r = b2.reshape(1, H)
    f3br = fc3_b.reshape(1, H)
    f4br = fc4_b.reshape(1, H)
    n2wr = norm2_w.reshape(1, H)
    n2br = norm2_b.reshape(1, H)
    fbr = fcf_b.reshape(1, 1)

    degp = _deg_kernel(dstp, z_stripe)
    d3 = degp.reshape(2 * GRID, 1, BN)

    dinv, xs = pl.pallas_call(
        _k2_body,
        grid=(GRID,),
        in_specs=[
            _row_spec(16),
            pl.BlockSpec((1, 1, BN), lambda i: (i, 0, 0)),
            pl.BlockSpec((1, 1, BN), lambda i: (GRID + i, 0, 0)),
        ],
        out_specs=[_row_spec(1), _row_spec(16)],
        out_shape=[
            jax.ShapeDtypeStruct((NPAD, 1), f32),
            jax.ShapeDtypeStruct((NPAD, 16), f32),
        ],
    )(x16, d3, d3)

    aggp = _agg16_kernel(srcp, dstp, xs, z16)

    h2, st1 = pl.pallas_call(
        _k4_body,
        grid=(GRID,),
        in_specs=[
            _part_spec(16, 0), _part_spec(16, 1),
            _row_spec(16), _row_spec(1),
            _const_spec((16, H)), _const_spec((1, H)),
            _const_spec((H, H)), _const_spec((1, H)),
            _const_spec((H, H)), _const_spec((1, H)),
        ],
        out_specs=[_row_spec(H), _const_spec((1, 128))],
        out_shape=[
            jax.ShapeDtypeStruct((NPAD, H), f32),
            jax.ShapeDtypeStruct((1, 128), f32),
        ],
    )(aggp, aggp, xs, dinv, w1p, b1r, fc1_w, f1br, fc2_w, f2br)

    hn, g0, g1, g2, g3 = pl.pallas_call(
        _k5_body,
        grid=(GRID,),
        in_specs=[
            _row_spec(H), _const_spec((1, 128)),
            _const_spec((1, H)), _const_spec((1, H)), _row_spec(1),
        ],
        out_specs=[_row_spec(H)] + [_row_spec(16)] * 4,
        out_shape=[jax.ShapeDtypeStruct((NPAD, H), f32)]
        + [jax.ShapeDtypeStruct((NPAD, 16), f32)] * 4,
    )(h2, st1, n1wr, n1br, dinv)

    a2 = [_agg16_kernel(srcp, dstp, gq, z16) for gq in (g0, g1, g2, g3)]

    v, st2 = pl.pallas_call(
        _k7_body,
        grid=(GRID,),
        in_specs=[
            _part_spec(16, 0), _part_spec(16, 1),
            _part_spec(16, 0), _part_spec(16, 1),
            _part_spec(16, 0), _part_spec(16, 1),
            _part_spec(16, 0), _part_spec(16, 1),
            _row_spec(H), _row_spec(1),
            _const_spec((H, H)), _const_spec((1, H)),
            _const_spec((H, H)), _const_spec((1, H)),
            _const_spec((H, H)), _const_spec((1, H)),
        ],
        out_specs=[_row_spec(H), _const_spec((1, 128))],
        out_shape=[
            jax.ShapeDtypeStruct((NPAD, H), f32),
            jax.ShapeDtypeStruct((1, 128), f32),
        ],
    )(a2[0], a2[0], a2[1], a2[1], a2[2], a2[2], a2[3], a2[3],
      hn, dinv, W2, b2r, fc3_w, f3br, fc4_w, f4br)

    out = pl.pallas_call(
        _k8_body,
        grid=(GRID,),
        in_specs=[
            _row_spec(H), _const_spec((1, 128)),
            _const_spec((1, H)), _const_spec((1, H)),
            _const_spec((H, 1)), _const_spec((1, 1)),
            _row_spec(1),
        ],
        out_specs=pl.BlockSpec((G, 1), lambda i: (0, 0)),
        out_shape=jax.ShapeDtypeStruct((G, 1), f32),
        scratch_shapes=[pltpu.VMEM((G, 2), f32)],
    )(v, st2, n2wr, n2br, fcf_w, fbr, batchp)

    return out
